# baseline (speedup 1.0000x reference)
import functools

import numpy as np

import jax
import jax.numpy as jnp
from jax import lax
from jax.experimental import pallas as pl
from jax.experimental.pallas import tpu as pltpu
from jax.experimental.pallas import tpu_sc as plsc

R = 16
NUM_ROWS = 31
ROW_DIM = 16
F = 16
HID = 512
MLP_IN = R * F

_KEY_BASE = np.uint32(0x38D00000)


def _prep_kernel(obs_hbm, keys_ref, robots_ref, buf, sems, *, envs_per_step):
    E = envs_per_step
    i = pl.program_id(0)

    def _copies(step, slot):
        return [
            pltpu.make_async_copy(
                obs_hbm.at[pl.ds((E * step + k) * R, 1), :],
                buf.at[slot, pl.ds(k, 1), :], sems.at[slot])
            for k in range(E)
        ]

    @pl.when(i == 0)
    def _prologue():
        for c in _copies(0, 0) + _copies(1, 1):
            c.start()

    @pl.when(i + 2 < pl.num_programs(0))
    def _prefetch():
        for c in _copies(i + 2, (i + 2) % 3):
            c.start()

    slot = i % 3
    for c in _copies(i, slot):
        c.wait()
    robots = jnp.stack(
        [buf[slot, :, pl.ds(NUM_ROWS * ROW_DIM * r, F)]
         for r in range(R)], axis=1)
    robots_ref[...] = robots.reshape(E, R * F)
    px = robots[:, :, F - 2]
    py = robots[:, :, F - 1]
    lane = lax.broadcasted_iota(jnp.int32, (R, R * R), 1)
    row = lax.broadcasted_iota(jnp.int32, (R, R * R), 0)
    rep = ((lane // R) == row).astype(jnp.float32)
    ax = jnp.concatenate([px] * R, axis=1)
    ay = jnp.concatenate([py] * R, axis=1)
    bx = jnp.dot(px, rep, preferred_element_type=jnp.float32,
                 precision=lax.Precision.HIGHEST)
    by = jnp.dot(py, rep, preferred_element_type=jnp.float32,
                 precision=lax.Precision.HIGHEST)
    dx = ax - bx
    dy = ay - by
    d2 = dx * dx + dy * dy
    dist = jnp.sqrt(d2 + 1e-08)
    bits = lax.bitcast_convert_type(dist, jnp.uint32)
    o_io = (lax.broadcasted_iota(jnp.uint32, (1, R * R), 1)
            % np.uint32(R))
    keys_ref[...] = (bits - _KEY_BASE) * np.uint32(16) + o_io


def _sc_body(keys_hbm, robots_hbm, out_hbm, kbuf, rbuf, obuf, sem, *,
             rows_per_worker, envs_per_worker, num_cores):
    wid = lax.axis_index("s") * num_cores + lax.axis_index("c")
    row0 = wid * rows_per_worker
    env0 = wid * envs_per_worker
    cks = pltpu.make_async_copy(keys_hbm.at[pl.ds(env0, envs_per_worker)],
                                kbuf, sem)
    crb = pltpu.make_async_copy(robots_hbm.at[pl.ds(env0, envs_per_worker)],
                                rbuf, sem)
    cks.start()
    crb.start()
    cks.wait()
    crb.wait()
    io16 = lax.iota(jnp.int32, R)
    ksplats = [jnp.full((R,), k, jnp.int32) for k in range(R)]
    UNROLL = 4

    def _rows(r4, carry):
        for u in range(UNROLL):
            r = r4 * UNROLL + u
            keys = kbuf[r // R, pl.ds((r % R) * R, R)]
            _, order = plsc.sort_key_val(keys, io16)
            e_splat = jnp.full((R,), r // R, jnp.int32)
            for k in range(R):
                ok = order.at[ksplats[k]].get(mode="promise_in_bounds")
                row = plsc.load_gather(rbuf, [e_splat, ok * F + io16])
                obuf[r, pl.ds(k * F, F)] = row
        return carry

    lax.fori_loop(0, rows_per_worker // UNROLL, _rows, 0)
    pltpu.sync_copy(obuf, out_hbm.at[pl.ds(row0, rows_per_worker)])


def _mlp_kernel(x_ref, fn_g_ref, fn_b_ref, w0t_ref, b0_ref, g0_ref, be0_ref,
                w1t_ref, b1_ref, g1_ref, be1_ref, wvt_ref, bv_ref, out_ref):
    def _ln(v, g, b):
        m = v.mean(axis=-1, keepdims=True)
        var = ((v - m) ** 2).mean(axis=-1, keepdims=True)
        return (v - m) / jnp.sqrt(var + 1e-5) * g + b

    x = _ln(x_ref[...], fn_g_ref[...], fn_b_ref[...])
    h = jnp.maximum(jnp.dot(x, w0t_ref[...], preferred_element_type=jnp.float32)
                    + b0_ref[...], 0.0)
    h = _ln(h, g0_ref[...], be0_ref[...])
    h = jnp.maximum(jnp.dot(h, w1t_ref[...], preferred_element_type=jnp.float32)
                    + b1_ref[...], 0.0)
    h = _ln(h, g1_ref[...], be1_ref[...])
    out_ref[...] = jnp.dot(h, wvt_ref[...],
                           preferred_element_type=jnp.float32) + bv_ref[...]


def kernel(cent_obs, rnn_states, masks, fn_g, fn_b, W0, b0, g0, be0, W1, b1,
           g1, be1, Wv, bv):
    batch = cent_obs.shape[0]
    n_envs = batch // R
    IN_DIM = cent_obs.shape[1]

    E = 16
    keys3, robots_flat = pl.pallas_call(
        functools.partial(_prep_kernel, envs_per_step=E),
        grid=(n_envs // E,),
        in_specs=[pl.BlockSpec(memory_space=pltpu.MemorySpace.HBM)],
        out_specs=[
            pl.BlockSpec((E, R * R), lambda i: (i, 0)),
            pl.BlockSpec((E, MLP_IN), lambda i: (i, 0)),
        ],
        out_shape=[
            jax.ShapeDtypeStruct((n_envs, R * R), jnp.uint32),
            jax.ShapeDtypeStruct((n_envs, MLP_IN), jnp.float32),
        ],
        scratch_shapes=[
            pltpu.VMEM((3, E, IN_DIM), jnp.float32),
            pltpu.SemaphoreType.DMA((3,)),
        ],
        compiler_params=pltpu.CompilerParams(
            dimension_semantics=("arbitrary",)),
    )(cent_obs)

    NUM_CORES, NUM_SUBCORES = 2, 16
    n_workers = NUM_CORES * NUM_SUBCORES
    scmesh = plsc.VectorSubcoreMesh(core_axis_name="c", subcore_axis_name="s")

    def _vec(v):
        return v.reshape(1, -1)

    full = lambda shape: pl.BlockSpec(shape, lambda i: (0,) * len(shape))

    def _sc_half(keys_h, robots_h, batch_h, envs_h):
        rows_pw = batch_h // n_workers
        envs_pw = envs_h // n_workers
        sc_fn = pl.kernel(
            functools.partial(_sc_body, rows_per_worker=rows_pw,
                              envs_per_worker=envs_pw, num_cores=NUM_CORES),
            mesh=scmesh,
            out_type=jax.ShapeDtypeStruct((batch_h, MLP_IN), jnp.float32),
            scratch_types=[
                pltpu.VMEM((envs_pw, R * R), jnp.uint32),
                pltpu.VMEM((envs_pw, MLP_IN), jnp.float32),
                pltpu.VMEM((rows_pw, MLP_IN), jnp.float32),
                pltpu.SemaphoreType.DMA,
            ],
            compiler_params=pltpu.CompilerParams(needs_layout_passes=False),
        )
        return sc_fn(keys_h, robots_h)

    def _mlp_half(critic_h, batch_h):
        ROWS = 1024
        return pl.pallas_call(
            _mlp_kernel,
            grid=(batch_h // ROWS,),
            in_specs=[
                pl.BlockSpec((ROWS, MLP_IN), lambda i: (i, 0)),
                full((1, MLP_IN)), full((1, MLP_IN)),
                full((MLP_IN, HID)), full((1, HID)), full((1, HID)),
                full((1, HID)),
                full((HID, HID)), full((1, HID)), full((1, HID)),
                full((1, HID)),
                full((HID, 1)), full((1, 1)),
            ],
            out_specs=pl.BlockSpec((ROWS, 1), lambda i: (i, 0)),
            out_shape=jax.ShapeDtypeStruct((batch_h, 1), jnp.float32),
            compiler_params=pltpu.CompilerParams(
                dimension_semantics=("parallel",)),
        )(critic_h, _vec(fn_g), _vec(fn_b), W0.T, _vec(b0), _vec(g0),
          _vec(be0), W1.T, _vec(b1), _vec(g1), _vec(be1), Wv.T, _vec(bv))

    half_e = n_envs // 2
    half_b = batch // 2
    critic0 = _sc_half(keys3[:half_e], robots_flat[:half_e], half_b, half_e)
    critic1 = _sc_half(keys3[half_e:], robots_flat[half_e:], half_b, half_e)
    values0 = _mlp_half(critic0, half_b)
    values1 = _mlp_half(critic1, half_b)
    values = jnp.concatenate([values0, values1], axis=0)
    return (values, rnn_states)


# speedup vs baseline: 1.0489x; 1.0489x over previous
import functools

import numpy as np

import jax
import jax.numpy as jnp
from jax import lax
from jax.experimental import pallas as pl
from jax.experimental.pallas import tpu as pltpu
from jax.experimental.pallas import tpu_sc as plsc

R = 16
NUM_ROWS = 31
ROW_DIM = 16
F = 16
HID = 512
MLP_IN = R * F

_KEY_BASE = np.uint32(0x38D00000)


def _prep_kernel(obs_hbm, keys_ref, robots_ref, buf, sems, *, envs_per_step):
    E = envs_per_step
    i = pl.program_id(0)

    def _copies(step, slot):
        return [
            pltpu.make_async_copy(
                obs_hbm.at[pl.ds((E * step + k) * R, 1), :],
                buf.at[slot, pl.ds(k, 1), :], sems.at[slot])
            for k in range(E)
        ]

    @pl.when(i == 0)
    def _prologue():
        for c in _copies(0, 0) + _copies(1, 1):
            c.start()

    @pl.when(i + 2 < pl.num_programs(0))
    def _prefetch():
        for c in _copies(i + 2, (i + 2) % 3):
            c.start()

    slot = i % 3
    for c in _copies(i, slot):
        c.wait()
    robots = jnp.stack(
        [buf[slot, :, pl.ds(NUM_ROWS * ROW_DIM * r, F)]
         for r in range(R)], axis=1)
    robots_ref[...] = robots.reshape(E, R * F)
    px = robots[:, :, F - 2]
    py = robots[:, :, F - 1]
    lane = lax.broadcasted_iota(jnp.int32, (R, R * R), 1)
    row = lax.broadcasted_iota(jnp.int32, (R, R * R), 0)
    rep = ((lane // R) == row).astype(jnp.float32)
    ax = jnp.concatenate([px] * R, axis=1)
    ay = jnp.concatenate([py] * R, axis=1)
    bx = jnp.dot(px, rep, preferred_element_type=jnp.float32,
                 precision=lax.Precision.HIGHEST)
    by = jnp.dot(py, rep, preferred_element_type=jnp.float32,
                 precision=lax.Precision.HIGHEST)
    dx = ax - bx
    dy = ay - by
    d2 = dx * dx + dy * dy
    dist = jnp.sqrt(d2 + 1e-08)
    bits = lax.bitcast_convert_type(dist, jnp.uint32)
    o_io = (lax.broadcasted_iota(jnp.uint32, (1, R * R), 1)
            % np.uint32(R))
    keys_ref[...] = (bits - _KEY_BASE) * np.uint32(16) + o_io


def _sc_body(keys_hbm, robots_hbm, out_hbm, kbuf, rbuf, obuf, sem, *,
             rows_per_worker, envs_per_worker, num_cores):
    wid = lax.axis_index("s") * num_cores + lax.axis_index("c")
    row0 = wid * rows_per_worker
    env0 = wid * envs_per_worker
    cks = pltpu.make_async_copy(keys_hbm.at[pl.ds(env0, envs_per_worker)],
                                kbuf, sem)
    crb = pltpu.make_async_copy(robots_hbm.at[pl.ds(env0, envs_per_worker)],
                                rbuf, sem)
    cks.start()
    crb.start()
    cks.wait()
    crb.wait()
    io16 = lax.iota(jnp.int32, R)
    ksplats = [jnp.full((R,), k, jnp.int32) for k in range(R)]
    UNROLL = 4

    def _rows(r4, carry):
        for u in range(UNROLL):
            r = r4 * UNROLL + u
            keys = kbuf[r // R, pl.ds((r % R) * R, R)]
            _, order = plsc.sort_key_val(keys, io16)
            e_splat = jnp.full((R,), r // R, jnp.int32)
            for k in range(R):
                ok = order.at[ksplats[k]].get(mode="promise_in_bounds")
                row = plsc.load_gather(rbuf, [e_splat, ok * F + io16])
                obuf[r, pl.ds(k * F, F)] = row
        return carry

    lax.fori_loop(0, rows_per_worker // UNROLL, _rows, 0)
    pltpu.sync_copy(obuf, out_hbm.at[pl.ds(row0, rows_per_worker)])


def _mlp_kernel(x_ref, fn_g_ref, fn_b_ref, w0t_ref, b0_ref, g0_ref, be0_ref,
                w1t_ref, b1_ref, g1_ref, be1_ref, wvt_ref, bv_ref, out_ref):
    def _ln(v, g, b):
        m = v.mean(axis=-1, keepdims=True)
        var = ((v - m) ** 2).mean(axis=-1, keepdims=True)
        return (v - m) / jnp.sqrt(var + 1e-5) * g + b

    x = _ln(x_ref[...], fn_g_ref[...], fn_b_ref[...])
    h = jnp.maximum(jnp.dot(x, w0t_ref[...], preferred_element_type=jnp.float32)
                    + b0_ref[...], 0.0)
    h = _ln(h, g0_ref[...], be0_ref[...])
    h = jnp.maximum(jnp.dot(h, w1t_ref[...], preferred_element_type=jnp.float32)
                    + b1_ref[...], 0.0)
    h = _ln(h, g1_ref[...], be1_ref[...])
    out_ref[...] = jnp.dot(h, wvt_ref[...],
                           preferred_element_type=jnp.float32) + bv_ref[...]


def kernel(cent_obs, rnn_states, masks, fn_g, fn_b, W0, b0, g0, be0, W1, b1,
           g1, be1, Wv, bv):
    batch = cent_obs.shape[0]
    n_envs = batch // R
    IN_DIM = cent_obs.shape[1]

    E = 16
    keys3, robots_flat = pl.pallas_call(
        functools.partial(_prep_kernel, envs_per_step=E),
        grid=(n_envs // E,),
        in_specs=[pl.BlockSpec(memory_space=pltpu.MemorySpace.HBM)],
        out_specs=[
            pl.BlockSpec((E, R * R), lambda i: (i, 0)),
            pl.BlockSpec((E, MLP_IN), lambda i: (i, 0)),
        ],
        out_shape=[
            jax.ShapeDtypeStruct((n_envs, R * R), jnp.uint32),
            jax.ShapeDtypeStruct((n_envs, MLP_IN), jnp.float32),
        ],
        scratch_shapes=[
            pltpu.VMEM((3, E, IN_DIM), jnp.float32),
            pltpu.SemaphoreType.DMA((3,)),
        ],
        compiler_params=pltpu.CompilerParams(
            dimension_semantics=("arbitrary",)),
    )(cent_obs)

    NUM_CORES, NUM_SUBCORES = 2, 16
    n_workers = NUM_CORES * NUM_SUBCORES
    scmesh = plsc.VectorSubcoreMesh(core_axis_name="c", subcore_axis_name="s")

    def _vec(v):
        return v.reshape(1, -1)

    full = lambda shape: pl.BlockSpec(shape, lambda i: (0,) * len(shape))

    def _sc_half(keys_h, robots_h, batch_h, envs_h):
        rows_pw = batch_h // n_workers
        envs_pw = envs_h // n_workers
        sc_fn = pl.kernel(
            functools.partial(_sc_body, rows_per_worker=rows_pw,
                              envs_per_worker=envs_pw, num_cores=NUM_CORES),
            mesh=scmesh,
            out_type=jax.ShapeDtypeStruct((batch_h, MLP_IN), jnp.float32),
            scratch_types=[
                pltpu.VMEM((envs_pw, R * R), jnp.uint32),
                pltpu.VMEM((envs_pw, MLP_IN), jnp.float32),
                pltpu.VMEM((rows_pw, MLP_IN), jnp.float32),
                pltpu.SemaphoreType.DMA,
            ],
            compiler_params=pltpu.CompilerParams(needs_layout_passes=False),
        )
        return sc_fn(keys_h, robots_h)

    def _mlp_half(critic_h, batch_h):
        ROWS = 1024
        return pl.pallas_call(
            _mlp_kernel,
            grid=(batch_h // ROWS,),
            in_specs=[
                pl.BlockSpec((ROWS, MLP_IN), lambda i: (i, 0)),
                full((1, MLP_IN)), full((1, MLP_IN)),
                full((MLP_IN, HID)), full((1, HID)), full((1, HID)),
                full((1, HID)),
                full((HID, HID)), full((1, HID)), full((1, HID)),
                full((1, HID)),
                full((HID, 1)), full((1, 1)),
            ],
            out_specs=pl.BlockSpec((ROWS, 1), lambda i: (i, 0)),
            out_shape=jax.ShapeDtypeStruct((batch_h, 1), jnp.float32),
            compiler_params=pltpu.CompilerParams(
                dimension_semantics=("parallel",)),
        )(critic_h, _vec(fn_g), _vec(fn_b), W0.T, _vec(b0), _vec(g0),
          _vec(be0), W1.T, _vec(b1), _vec(g1), _vec(be1), Wv.T, _vec(bv))

    critic = _sc_half(keys3, robots_flat, batch, n_envs)
    values = _mlp_half(critic, batch)
    return (values, rnn_states)


# speedup vs baseline: 1.0690x; 1.0191x over previous
import functools

import numpy as np

import jax
import jax.numpy as jnp
from jax import lax
from jax.experimental import pallas as pl
from jax.experimental.pallas import tpu as pltpu
from jax.experimental.pallas import tpu_sc as plsc

R = 16
NUM_ROWS = 31
ROW_DIM = 16
F = 16
HID = 512
MLP_IN = R * F

_KEY_BASE = np.uint32(0x38D00000)


def _prep_kernel(obs_hbm, keys_ref, robots_ref, buf, sems, *, envs_per_step):
    E = envs_per_step
    i = pl.program_id(0)

    def _copies(step, slot):
        return [
            pltpu.make_async_copy(
                obs_hbm.at[pl.ds((E * step + k) * R, 1), :],
                buf.at[slot, pl.ds(k, 1), :], sems.at[slot])
            for k in range(E)
        ]

    @pl.when(i == 0)
    def _prologue():
        for c in _copies(0, 0) + _copies(1, 1):
            c.start()

    @pl.when(i + 2 < pl.num_programs(0))
    def _prefetch():
        for c in _copies(i + 2, (i + 2) % 3):
            c.start()

    slot = i % 3
    for c in _copies(i, slot):
        c.wait()
    robots = jnp.stack(
        [buf[slot, :, pl.ds(NUM_ROWS * ROW_DIM * r, F)]
         for r in range(R)], axis=1)
    robots_ref[...] = robots.reshape(E, R * F)
    px = robots[:, :, F - 2]
    py = robots[:, :, F - 1]
    lane = lax.broadcasted_iota(jnp.int32, (R, R * R), 1)
    row = lax.broadcasted_iota(jnp.int32, (R, R * R), 0)
    rep = ((lane // R) == row).astype(jnp.float32)
    ax = jnp.concatenate([px] * R, axis=1)
    ay = jnp.concatenate([py] * R, axis=1)
    bx = jnp.dot(px, rep, preferred_element_type=jnp.float32,
                 precision=lax.Precision.HIGHEST)
    by = jnp.dot(py, rep, preferred_element_type=jnp.float32,
                 precision=lax.Precision.HIGHEST)
    dx = ax - bx
    dy = ay - by
    d2 = dx * dx + dy * dy
    dist = jnp.sqrt(d2 + 1e-08)
    bits = lax.bitcast_convert_type(dist, jnp.uint32)
    o_io = (lax.broadcasted_iota(jnp.uint32, (1, R * R), 1)
            % np.uint32(R))
    keys_ref[...] = (bits - _KEY_BASE) * np.uint32(16) + o_io


def _sc_body(keys_hbm, robots_hbm, out_hbm, kbuf, rbuf, obuf, sem, *,
             rows_per_worker, envs_per_worker, num_cores):
    wid = lax.axis_index("s") * num_cores + lax.axis_index("c")
    row0 = wid * rows_per_worker
    env0 = wid * envs_per_worker
    cks = pltpu.make_async_copy(keys_hbm.at[pl.ds(env0, envs_per_worker)],
                                kbuf, sem)
    crb = pltpu.make_async_copy(robots_hbm.at[pl.ds(env0, envs_per_worker)],
                                rbuf, sem)
    cks.start()
    crb.start()
    cks.wait()
    crb.wait()
    io16 = lax.iota(jnp.int32, R)
    ksplats = [jnp.full((R,), k, jnp.int32) for k in range(R)]
    UNROLL = 4

    def _rows(r4, carry):
        for u in range(UNROLL):
            r = r4 * UNROLL + u
            keys = kbuf[r // R, pl.ds((r % R) * R, R)]
            _, order = plsc.sort_key_val(keys, io16)
            e_splat = jnp.full((R,), r // R, jnp.int32)
            for k in range(R):
                ok = order.at[ksplats[k]].get(mode="promise_in_bounds")
                row = plsc.load_gather(rbuf, [e_splat, ok * F + io16])
                obuf[r, pl.ds(k * F, F)] = row
        return carry

    lax.fori_loop(0, rows_per_worker // UNROLL, _rows, 0)
    pltpu.sync_copy(obuf, out_hbm.at[pl.ds(row0, rows_per_worker)])


def _mlp_kernel(x_ref, fn_g_ref, fn_b_ref, w0t_ref, b0_ref, g0_ref, be0_ref,
                w1t_ref, b1_ref, g1_ref, be1_ref, wvt_ref, bv_ref, out_ref):
    def _ln(v, g, b):
        m = v.mean(axis=-1, keepdims=True)
        var = ((v - m) ** 2).mean(axis=-1, keepdims=True)
        return (v - m) / jnp.sqrt(var + 1e-5) * g + b

    x = _ln(x_ref[...], fn_g_ref[...], fn_b_ref[...])
    h = jnp.maximum(jnp.dot(x, w0t_ref[...], preferred_element_type=jnp.float32)
                    + b0_ref[...], 0.0)
    h = _ln(h, g0_ref[...], be0_ref[...])
    h = jnp.maximum(jnp.dot(h, w1t_ref[...], preferred_element_type=jnp.float32)
                    + b1_ref[...], 0.0)
    h = _ln(h, g1_ref[...], be1_ref[...])
    out_ref[...] = jnp.dot(h, wvt_ref[...],
                           preferred_element_type=jnp.float32) + bv_ref[...]


def kernel(cent_obs, rnn_states, masks, fn_g, fn_b, W0, b0, g0, be0, W1, b1,
           g1, be1, Wv, bv):
    batch = cent_obs.shape[0]
    n_envs = batch // R
    IN_DIM = cent_obs.shape[1]

    E = 32
    keys3, robots_flat = pl.pallas_call(
        functools.partial(_prep_kernel, envs_per_step=E),
        grid=(n_envs // E,),
        in_specs=[pl.BlockSpec(memory_space=pltpu.MemorySpace.HBM)],
        out_specs=[
            pl.BlockSpec((E, R * R), lambda i: (i, 0)),
            pl.BlockSpec((E, MLP_IN), lambda i: (i, 0)),
        ],
        out_shape=[
            jax.ShapeDtypeStruct((n_envs, R * R), jnp.uint32),
            jax.ShapeDtypeStruct((n_envs, MLP_IN), jnp.float32),
        ],
        scratch_shapes=[
            pltpu.VMEM((3, E, IN_DIM), jnp.float32),
            pltpu.SemaphoreType.DMA((3,)),
        ],
        compiler_params=pltpu.CompilerParams(
            dimension_semantics=("arbitrary",)),
    )(cent_obs)

    NUM_CORES, NUM_SUBCORES = 2, 16
    n_workers = NUM_CORES * NUM_SUBCORES
    scmesh = plsc.VectorSubcoreMesh(core_axis_name="c", subcore_axis_name="s")

    def _vec(v):
        return v.reshape(1, -1)

    full = lambda shape: pl.BlockSpec(shape, lambda i: (0,) * len(shape))

    def _sc_half(keys_h, robots_h, batch_h, envs_h):
        rows_pw = batch_h // n_workers
        envs_pw = envs_h // n_workers
        sc_fn = pl.kernel(
            functools.partial(_sc_body, rows_per_worker=rows_pw,
                              envs_per_worker=envs_pw, num_cores=NUM_CORES),
            mesh=scmesh,
            out_type=jax.ShapeDtypeStruct((batch_h, MLP_IN), jnp.float32),
            scratch_types=[
                pltpu.VMEM((envs_pw, R * R), jnp.uint32),
                pltpu.VMEM((envs_pw, MLP_IN), jnp.float32),
                pltpu.VMEM((rows_pw, MLP_IN), jnp.float32),
                pltpu.SemaphoreType.DMA,
            ],
            compiler_params=pltpu.CompilerParams(needs_layout_passes=False),
        )
        return sc_fn(keys_h, robots_h)

    def _mlp_half(critic_h, batch_h):
        ROWS = 1024
        return pl.pallas_call(
            _mlp_kernel,
            grid=(batch_h // ROWS,),
            in_specs=[
                pl.BlockSpec((ROWS, MLP_IN), lambda i: (i, 0)),
                full((1, MLP_IN)), full((1, MLP_IN)),
                full((MLP_IN, HID)), full((1, HID)), full((1, HID)),
                full((1, HID)),
                full((HID, HID)), full((1, HID)), full((1, HID)),
                full((1, HID)),
                full((HID, 1)), full((1, 1)),
            ],
            out_specs=pl.BlockSpec((ROWS, 1), lambda i: (i, 0)),
            out_shape=jax.ShapeDtypeStruct((batch_h, 1), jnp.float32),
            compiler_params=pltpu.CompilerParams(
                dimension_semantics=("parallel",)),
        )(critic_h, _vec(fn_g), _vec(fn_b), W0.T, _vec(b0), _vec(g0),
          _vec(be0), W1.T, _vec(b1), _vec(g1), _vec(be1), Wv.T, _vec(bv))

    critic = _sc_half(keys3, robots_flat, batch, n_envs)
    values = _mlp_half(critic, batch)
    return (values, rnn_states)


# speedup vs baseline: 1.0821x; 1.0123x over previous
import functools

import numpy as np

import jax
import jax.numpy as jnp
from jax import lax
from jax.experimental import pallas as pl
from jax.experimental.pallas import tpu as pltpu
from jax.experimental.pallas import tpu_sc as plsc

R = 16
NUM_ROWS = 31
ROW_DIM = 16
F = 16
HID = 512
MLP_IN = R * F

_KEY_BASE = np.uint32(0x38D00000)


def _prep_kernel(obs_hbm, keys_ref, robots_ref, buf, sems, *, envs_per_step):
    E = envs_per_step
    i = pl.program_id(0)

    def _copies(step, slot):
        return [
            pltpu.make_async_copy(
                obs_hbm.at[pl.ds((E * step + k) * R, 1), :],
                buf.at[slot, pl.ds(k, 1), :], sems.at[slot])
            for k in range(E)
        ]

    @pl.when(i == 0)
    def _prologue():
        for c in _copies(0, 0) + _copies(1, 1):
            c.start()

    @pl.when(i + 2 < pl.num_programs(0))
    def _prefetch():
        for c in _copies(i + 2, (i + 2) % 3):
            c.start()

    slot = i % 3
    for c in _copies(i, slot):
        c.wait()
    robots = jnp.stack(
        [buf[slot, :, pl.ds(NUM_ROWS * ROW_DIM * r, F)]
         for r in range(R)], axis=1)
    robots_ref[...] = robots.reshape(E, R * F)
    px = robots[:, :, F - 2]
    py = robots[:, :, F - 1]
    lane = lax.broadcasted_iota(jnp.int32, (R, R * R), 1)
    row = lax.broadcasted_iota(jnp.int32, (R, R * R), 0)
    rep = ((lane // R) == row).astype(jnp.float32)
    ax = jnp.concatenate([px] * R, axis=1)
    ay = jnp.concatenate([py] * R, axis=1)
    bx = jnp.dot(px, rep, preferred_element_type=jnp.float32,
                 precision=lax.Precision.HIGHEST)
    by = jnp.dot(py, rep, preferred_element_type=jnp.float32,
                 precision=lax.Precision.HIGHEST)
    dx = ax - bx
    dy = ay - by
    d2 = dx * dx + dy * dy
    dist = jnp.sqrt(d2 + 1e-08)
    bits = lax.bitcast_convert_type(dist, jnp.uint32)
    o_io = (lax.broadcasted_iota(jnp.uint32, (1, R * R), 1)
            % np.uint32(R))
    keys_ref[...] = (bits - _KEY_BASE) * np.uint32(16) + o_io


def _sc_body(keys_hbm, robots_hbm, out_hbm, kbuf, rbuf, obuf, sem, *,
             rows_per_worker, envs_per_worker, num_cores):
    wid = lax.axis_index("s") * num_cores + lax.axis_index("c")
    row0 = wid * rows_per_worker
    env0 = wid * envs_per_worker
    cks = pltpu.make_async_copy(keys_hbm.at[pl.ds(env0, envs_per_worker)],
                                kbuf, sem)
    crb = pltpu.make_async_copy(robots_hbm.at[pl.ds(env0, envs_per_worker)],
                                rbuf, sem)
    cks.start()
    crb.start()
    cks.wait()
    crb.wait()
    io16 = lax.iota(jnp.int32, R)
    ksplats = [jnp.full((R,), k, jnp.int32) for k in range(R)]
    UNROLL = 4

    def _rows(r4, carry):
        for u in range(UNROLL):
            r = r4 * UNROLL + u
            keys = kbuf[r // R, pl.ds((r % R) * R, R)]
            _, order = plsc.sort_key_val(keys, io16)
            e_splat = jnp.full((R,), r // R, jnp.int32)
            for k in range(R):
                ok = order.at[ksplats[k]].get(mode="promise_in_bounds")
                row = plsc.load_gather(rbuf, [e_splat, ok * F + io16])
                obuf[r, pl.ds(k * F, F)] = row
        return carry

    lax.fori_loop(0, rows_per_worker // UNROLL, _rows, 0)
    pltpu.sync_copy(obuf, out_hbm.at[pl.ds(row0, rows_per_worker)])


def _mlp_kernel(x_ref, fn_g_ref, fn_b_ref, w0t_ref, b0_ref, g0_ref, be0_ref,
                w1t_ref, b1_ref, g1_ref, be1_ref, wvt_ref, bv_ref, out_ref):
    def _ln(v, g, b):
        m = v.mean(axis=-1, keepdims=True)
        var = ((v - m) ** 2).mean(axis=-1, keepdims=True)
        return (v - m) / jnp.sqrt(var + 1e-5) * g + b

    x = _ln(x_ref[...], fn_g_ref[...], fn_b_ref[...])
    h = jnp.maximum(jnp.dot(x, w0t_ref[...], preferred_element_type=jnp.float32)
                    + b0_ref[...], 0.0)
    h = _ln(h, g0_ref[...], be0_ref[...])
    h = jnp.maximum(jnp.dot(h, w1t_ref[...], preferred_element_type=jnp.float32)
                    + b1_ref[...], 0.0)
    h = _ln(h, g1_ref[...], be1_ref[...])
    out_ref[...] = jnp.dot(h, wvt_ref[...],
                           preferred_element_type=jnp.float32) + bv_ref[...]


def kernel(cent_obs, rnn_states, masks, fn_g, fn_b, W0, b0, g0, be0, W1, b1,
           g1, be1, Wv, bv):
    batch = cent_obs.shape[0]
    n_envs = batch // R
    IN_DIM = cent_obs.shape[1]

    E = 64
    keys3, robots_flat = pl.pallas_call(
        functools.partial(_prep_kernel, envs_per_step=E),
        grid=(n_envs // E,),
        in_specs=[pl.BlockSpec(memory_space=pltpu.MemorySpace.HBM)],
        out_specs=[
            pl.BlockSpec((E, R * R), lambda i: (i, 0)),
            pl.BlockSpec((E, MLP_IN), lambda i: (i, 0)),
        ],
        out_shape=[
            jax.ShapeDtypeStruct((n_envs, R * R), jnp.uint32),
            jax.ShapeDtypeStruct((n_envs, MLP_IN), jnp.float32),
        ],
        scratch_shapes=[
            pltpu.VMEM((3, E, IN_DIM), jnp.float32),
            pltpu.SemaphoreType.DMA((3,)),
        ],
        compiler_params=pltpu.CompilerParams(
            dimension_semantics=("arbitrary",)),
    )(cent_obs)

    NUM_CORES, NUM_SUBCORES = 2, 16
    n_workers = NUM_CORES * NUM_SUBCORES
    scmesh = plsc.VectorSubcoreMesh(core_axis_name="c", subcore_axis_name="s")

    def _vec(v):
        return v.reshape(1, -1)

    full = lambda shape: pl.BlockSpec(shape, lambda i: (0,) * len(shape))

    def _sc_half(keys_h, robots_h, batch_h, envs_h):
        rows_pw = batch_h // n_workers
        envs_pw = envs_h // n_workers
        sc_fn = pl.kernel(
            functools.partial(_sc_body, rows_per_worker=rows_pw,
                              envs_per_worker=envs_pw, num_cores=NUM_CORES),
            mesh=scmesh,
            out_type=jax.ShapeDtypeStruct((batch_h, MLP_IN), jnp.float32),
            scratch_types=[
                pltpu.VMEM((envs_pw, R * R), jnp.uint32),
                pltpu.VMEM((envs_pw, MLP_IN), jnp.float32),
                pltpu.VMEM((rows_pw, MLP_IN), jnp.float32),
                pltpu.SemaphoreType.DMA,
            ],
            compiler_params=pltpu.CompilerParams(needs_layout_passes=False),
        )
        return sc_fn(keys_h, robots_h)

    def _mlp_half(critic_h, batch_h):
        ROWS = 1024
        return pl.pallas_call(
            _mlp_kernel,
            grid=(batch_h // ROWS,),
            in_specs=[
                pl.BlockSpec((ROWS, MLP_IN), lambda i: (i, 0)),
                full((1, MLP_IN)), full((1, MLP_IN)),
                full((MLP_IN, HID)), full((1, HID)), full((1, HID)),
                full((1, HID)),
                full((HID, HID)), full((1, HID)), full((1, HID)),
                full((1, HID)),
                full((HID, 1)), full((1, 1)),
            ],
            out_specs=pl.BlockSpec((ROWS, 1), lambda i: (i, 0)),
            out_shape=jax.ShapeDtypeStruct((batch_h, 1), jnp.float32),
            compiler_params=pltpu.CompilerParams(
                dimension_semantics=("parallel",)),
        )(critic_h, _vec(fn_g), _vec(fn_b), W0.T, _vec(b0), _vec(g0),
          _vec(be0), W1.T, _vec(b1), _vec(g1), _vec(be1), Wv.T, _vec(bv))

    critic = _sc_half(keys3, robots_flat, batch, n_envs)
    values = _mlp_half(critic, batch)
    return (values, rnn_states)
